# Initial kernel scaffold; baseline (speedup 1.0000x reference)
import jax
import jax.numpy as jnp
from jax.experimental import pallas as pl


def kernel(x, unnormalized_widths, unnormalized_heights, unnormalized_derivatives):
    raise NotImplementedError("write your pallas kernel here")


# speedup vs baseline: 701.2602x; 701.2602x over previous
import numpy as np
import jax
import jax.numpy as jnp
from jax.experimental import pallas as pl
from jax.experimental.pallas import tpu as pltpu

_DIM = 16
_K = 8
_TB = 3.0
_MIN_BW = 1e-3
_MIN_BH = 1e-3
_MIN_D = 1e-3
_PAD_C = float(np.log(np.exp(1 - _MIN_D) - 1))

_LANES = 128
_REP = _LANES // _DIM
_BLOCK_ROWS = 2048


def _softmax_rows(u):
    m = jnp.max(u, axis=0, keepdims=True)
    e = jnp.exp(u - m)
    return e / jnp.sum(e, axis=0, keepdims=True)


def _edges_from(u, min_b):
    w = _MIN_BW * 0 + min_b + (1 - min_b * _K) * _softmax_rows(u)
    edges = [jnp.full((1, _LANES), -_TB, dtype=u.dtype)]
    acc = jnp.zeros((1, _LANES), dtype=u.dtype)
    for k in range(_K - 1):
        acc = acc + w[k : k + 1, :]
        edges.append(2 * _TB * acc - _TB)
    edges.append(jnp.full((1, _LANES), _TB, dtype=u.dtype))
    return edges


def _spline_body(uw_ref, uh_ref, ud_ref, x_ref, out_ref, lad_ref):
    f32 = jnp.float32
    ew = _edges_from(uw_ref[...], _MIN_BW)
    eh = _edges_from(uh_ref[...], _MIN_BH)
    widths = [ew[k + 1] - ew[k] for k in range(_K)]
    heights = [eh[k + 1] - eh[k] for k in range(_K)]
    rw = [1.0 / widths[k] for k in range(_K)]

    ud = ud_ref[...]
    pad = jnp.full((1, _LANES), _PAD_C, dtype=f32)
    ud_rows = [pad] + [ud[k : k + 1, :] for k in range(_K - 1)] + [pad]
    derivs = [_MIN_D + jnp.log1p(jnp.exp(u)) for u in ud_rows]

    x = x_ref[...]
    inside = (x >= -_TB) & (x <= _TB)
    x_in = jnp.clip(x, -_TB, _TB)

    g_cumw = jnp.broadcast_to(ew[0], x.shape)
    g_rw = jnp.broadcast_to(rw[0], x.shape)
    g_h = jnp.broadcast_to(heights[0], x.shape)
    g_cumh = jnp.broadcast_to(eh[0], x.shape)
    g_d = jnp.broadcast_to(derivs[0], x.shape)
    g_d1 = jnp.broadcast_to(derivs[1], x.shape)
    for j in range(1, _K):
        m = (x_in >= ew[j]).astype(f32)
        g_cumw = g_cumw + (ew[j] - ew[j - 1]) * m
        g_rw = g_rw + (rw[j] - rw[j - 1]) * m
        g_h = g_h + (heights[j] - heights[j - 1]) * m
        g_cumh = g_cumh + (eh[j] - eh[j - 1]) * m
        g_d = g_d + (derivs[j] - derivs[j - 1]) * m
        g_d1 = g_d1 + (derivs[j + 1] - derivs[j]) * m

    g_delta = g_h * g_rw
    theta = (x_in - g_cumw) * g_rw
    omt = 1.0 - theta
    tomt = theta * omt
    th2 = theta * theta
    num = g_h * (g_delta * th2 + g_d * tomt)
    den = g_delta + (g_d + g_d1 - 2.0 * g_delta) * tomt
    rden = 1.0 / den
    out_in = g_cumh + num * rden
    dnum = (g_delta * g_delta) * (g_d1 * th2 + 2.0 * g_delta * tomt + g_d * (omt * omt))
    lad_in = jnp.log(dnum * rden * rden)

    out_ref[...] = jnp.where(inside, out_in, x)
    lad_ref[...] = jnp.where(inside, lad_in, 0.0)


def kernel(x, unnormalized_widths, unnormalized_heights, unnormalized_derivatives):
    n, d = x.shape
    rows = n * d // _LANES
    x2 = x.reshape(rows, _LANES)
    uw_t = jnp.tile(unnormalized_widths.T, (1, _REP))
    uh_t = jnp.tile(unnormalized_heights.T, (1, _REP))
    ud_t = jnp.tile(unnormalized_derivatives.T, (1, _REP))

    grid = (rows // _BLOCK_ROWS,)
    out, lad = pl.pallas_call(
        _spline_body,
        grid=grid,
        in_specs=[
            pl.BlockSpec((_K, _LANES), lambda i: (0, 0)),
            pl.BlockSpec((_K, _LANES), lambda i: (0, 0)),
            pl.BlockSpec((_K - 1, _LANES), lambda i: (0, 0)),
            pl.BlockSpec((_BLOCK_ROWS, _LANES), lambda i: (i, 0)),
        ],
        out_specs=[
            pl.BlockSpec((_BLOCK_ROWS, _LANES), lambda i: (i, 0)),
            pl.BlockSpec((_BLOCK_ROWS, _LANES), lambda i: (i, 0)),
        ],
        out_shape=[
            jax.ShapeDtypeStruct((rows, _LANES), jnp.float32),
            jax.ShapeDtypeStruct((rows, _LANES), jnp.float32),
        ],
        compiler_params=pltpu.CompilerParams(
            dimension_semantics=("arbitrary",),
        ),
    )(uw_t, uh_t, ud_t, x2)
    return out.reshape(n, d), lad.reshape(n, d)


# speedup vs baseline: 733.0754x; 1.0454x over previous
import numpy as np
import jax
import jax.numpy as jnp
from jax.experimental import pallas as pl
from jax.experimental.pallas import tpu as pltpu

_DIM = 16
_K = 8
_TB = 3.0
_MIN_BW = 1e-3
_MIN_BH = 1e-3
_MIN_D = 1e-3
_PAD_C = float(np.log(np.exp(1 - _MIN_D) - 1))

_LANES = 128
_REP = _LANES // _DIM
_BLOCK_IN_ROWS = 8192
_CHUNK = _BLOCK_IN_ROWS // _REP


def _softmax_rows(u):
    m = jnp.max(u, axis=0, keepdims=True)
    e = jnp.exp(u - m)
    return e / jnp.sum(e, axis=0, keepdims=True)


def _edges_from(u, min_b):
    w = min_b + (1 - min_b * _K) * _softmax_rows(u)
    edges = [jnp.full((1, _LANES), -_TB, dtype=u.dtype)]
    acc = jnp.zeros((1, _LANES), dtype=u.dtype)
    for k in range(_K - 1):
        acc = acc + w[k : k + 1, :]
        edges.append(2 * _TB * acc - _TB)
    edges.append(jnp.full((1, _LANES), _TB, dtype=u.dtype))
    return edges


def _spline_body(uw_ref, uh_ref, ud_ref, x_ref, out_ref, lad_ref):
    f32 = jnp.float32
    ew = _edges_from(uw_ref[...], _MIN_BW)
    eh = _edges_from(uh_ref[...], _MIN_BH)
    widths = [ew[k + 1] - ew[k] for k in range(_K)]
    heights = [eh[k + 1] - eh[k] for k in range(_K)]
    rw = [1.0 / widths[k] for k in range(_K)]

    ud = ud_ref[...]
    pad = jnp.full((1, _LANES), _PAD_C, dtype=f32)
    ud_rows = [pad] + [ud[k : k + 1, :] for k in range(_K - 1)] + [pad]
    derivs = [_MIN_D + jnp.log1p(jnp.exp(u)) for u in ud_rows]

    xb = x_ref[...]
    x = jnp.concatenate(
        [xb[g * _CHUNK : (g + 1) * _CHUNK, :] for g in range(_REP)], axis=1
    )
    inside = (x >= -_TB) & (x <= _TB)
    x_in = jnp.clip(x, -_TB, _TB)

    g_cumw = jnp.broadcast_to(ew[0], x.shape)
    g_rw = jnp.broadcast_to(rw[0], x.shape)
    g_h = jnp.broadcast_to(heights[0], x.shape)
    g_cumh = jnp.broadcast_to(eh[0], x.shape)
    g_d = jnp.broadcast_to(derivs[0], x.shape)
    g_d1 = jnp.broadcast_to(derivs[1], x.shape)
    for j in range(1, _K):
        m = (x_in >= ew[j]).astype(f32)
        g_cumw = g_cumw + (ew[j] - ew[j - 1]) * m
        g_rw = g_rw + (rw[j] - rw[j - 1]) * m
        g_h = g_h + (heights[j] - heights[j - 1]) * m
        g_cumh = g_cumh + (eh[j] - eh[j - 1]) * m
        g_d = g_d + (derivs[j] - derivs[j - 1]) * m
        g_d1 = g_d1 + (derivs[j + 1] - derivs[j]) * m

    g_delta = g_h * g_rw
    theta = (x_in - g_cumw) * g_rw
    omt = 1.0 - theta
    tomt = theta * omt
    th2 = theta * theta
    num = g_h * (g_delta * th2 + g_d * tomt)
    den = g_delta + (g_d + g_d1 - 2.0 * g_delta) * tomt
    rden = 1.0 / den
    out_in = g_cumh + num * rden
    dnum = (g_delta * g_delta) * (g_d1 * th2 + 2.0 * g_delta * tomt + g_d * (omt * omt))
    lad_in = jnp.log(dnum * rden * rden)

    out_c = jnp.where(inside, out_in, x)
    lad_c = jnp.where(inside, lad_in, 0.0)
    out_ref[...] = jnp.concatenate(
        [out_c[:, g * _DIM : (g + 1) * _DIM] for g in range(_REP)], axis=0
    )
    lad_ref[...] = jnp.concatenate(
        [lad_c[:, g * _DIM : (g + 1) * _DIM] for g in range(_REP)], axis=0
    )


def kernel(x, unnormalized_widths, unnormalized_heights, unnormalized_derivatives):
    n, d = x.shape
    uw_t = jnp.tile(unnormalized_widths.T, (1, _REP))
    uh_t = jnp.tile(unnormalized_heights.T, (1, _REP))
    ud_t = jnp.tile(unnormalized_derivatives.T, (1, _REP))

    grid = (n // _BLOCK_IN_ROWS,)
    out, lad = pl.pallas_call(
        _spline_body,
        grid=grid,
        in_specs=[
            pl.BlockSpec((_K, _LANES), lambda i: (0, 0)),
            pl.BlockSpec((_K, _LANES), lambda i: (0, 0)),
            pl.BlockSpec((_K - 1, _LANES), lambda i: (0, 0)),
            pl.BlockSpec((_BLOCK_IN_ROWS, _DIM), lambda i: (i, 0)),
        ],
        out_specs=[
            pl.BlockSpec((_BLOCK_IN_ROWS, _DIM), lambda i: (i, 0)),
            pl.BlockSpec((_BLOCK_IN_ROWS, _DIM), lambda i: (i, 0)),
        ],
        out_shape=[
            jax.ShapeDtypeStruct((n, d), jnp.float32),
            jax.ShapeDtypeStruct((n, d), jnp.float32),
        ],
        compiler_params=pltpu.CompilerParams(
            dimension_semantics=("arbitrary",),
        ),
    )(uw_t, uh_t, ud_t, x)
    return out, lad
